# baseline (speedup 1.0000x reference)
import functools

import jax
import jax.numpy as jnp
from jax import lax
from jax.experimental import pallas as pl
from jax.experimental.pallas import tpu as pltpu
from jax.experimental.pallas import tpu_sc as plsc

N = 10000
E = 320000
HID = 64
N_LAYERS = 6

NC, NS = 2, 16
NW = NC * NS
CH = 128
EW = E // NW
NCHUNK = -(-EW // CH)
GAHEAD = 3
NBUF = 6
EWP = NCHUNK * CH
NTAB = 10240
RPT = NTAB // NS
DUMMY = N
DW = 16


def _make_agg(D, with_deg):
    out_type = [jax.ShapeDtypeStruct((NTAB, D), jnp.float32) for _ in range(NC)]
    scratch = [
        pltpu.VMEM((NCHUNK, CH), jnp.int32),
        pltpu.VMEM((NCHUNK, CH), jnp.int32),
        pltpu.VMEM((NBUF, CH, D), jnp.float32),
        pltpu.VMEM_SHARED((NTAB, D), jnp.float32),
        pltpu.SemaphoreType.DMA,
    ] + [pltpu.SemaphoreType.DMA] * (2 * NBUF)
    if with_deg:
        out_type += [jax.ShapeDtypeStruct((NTAB, DW), jnp.float32)
                     for _ in range(NC)]
        scratch += [
            pltpu.VMEM((CH, DW), jnp.float32),
            pltpu.VMEM_SHARED((NTAB, DW), jnp.float32),
            pltpu.SemaphoreType.DMA,
        ]
    mesh = plsc.VectorSubcoreMesh(core_axis_name="c", subcore_axis_name="s")

    def body(*refs):
        if with_deg:
            (tab, rowi, coli, z, zd, ones_h,
             out0, out1, dout0, dout1,
             rowi_v, coli_v, rows_v, acc, sem_in, *rest) = refs
            sg, ss = rest[:NBUF], rest[NBUF:2 * NBUF]
            ones_v, dacc, sem_d = rest[2 * NBUF:]
        else:
            (tab, rowi, coli, z,
             out0, out1,
             rowi_v, coli_v, rows_v, acc, sem_in, *rest) = refs
            sg, ss = rest[:NBUF], rest[NBUF:2 * NBUF]
        c = lax.axis_index("c")
        s = lax.axis_index("s")
        w = c * NS + s
        sl = pl.ds(s * RPT, RPT)
        stage = [pltpu.async_copy(z, acc.at[sl], sem_in),
                 pltpu.async_copy(rowi.at[w], rowi_v, sem_in),
                 pltpu.async_copy(coli.at[w], coli_v, sem_in)]
        if with_deg:
            stage.append(pltpu.async_copy(zd, dacc.at[sl], sem_in))
            stage.append(pltpu.async_copy(ones_h, ones_v, sem_in))
        for cp in stage:
            cp.wait()
        plsc.subcore_barrier()

        gath = [None] * NCHUNK
        scat = [None] * NCHUNK
        dscat = [None] * NCHUNK

        def fire_gather(j):
            gath[j] = pltpu.async_copy(tab.at[coli_v.at[j]],
                                       rows_v.at[j % NBUF], sg[j % NBUF])

        for j in range(min(GAHEAD, NCHUNK)):
            fire_gather(j)
        for j in range(NCHUNK):
            b = j % NBUF
            gath[j].wait()
            scat[j] = pltpu.async_copy(rows_v.at[b], acc.at[rowi_v.at[j]],
                                       ss[b], add=True)
            if with_deg:
                dscat[j] = pltpu.async_copy(ones_v, dacc.at[rowi_v.at[j]],
                                            sem_d, add=True)
                if j >= 4:
                    dscat[j - 4].wait()
            nxt = j + GAHEAD
            if nxt < NCHUNK:
                if nxt >= NBUF:
                    scat[nxt - NBUF].wait()
                fire_gather(nxt)
        for j in range(max(0, NCHUNK - NBUF), NCHUNK):
            scat[j].wait()
        if with_deg:
            for j in range(max(0, NCHUNK - 4), NCHUNK):
                dscat[j].wait()
        plsc.subcore_barrier()

        @pl.when(c == 0)
        def _():
            pltpu.sync_copy(acc.at[sl], out0.at[sl])
            if with_deg:
                pltpu.sync_copy(dacc.at[sl], dout0.at[sl])

        @pl.when(c == 1)
        def _():
            pltpu.sync_copy(acc.at[sl], out1.at[sl])
            if with_deg:
                pltpu.sync_copy(dacc.at[sl], dout1.at[sl])

    return pl.kernel(body, out_type=out_type, mesh=mesh,
                     scratch_types=scratch,
                     compiler_params=pltpu.CompilerParams(
                         use_tc_tiling_on_sc=False))


_agg64_deg = _make_agg(HID, True)
_agg64 = _make_agg(HID, False)
_agg16 = _make_agg(DW, False)


def _tc_matmul(x, W):
    def f(x_ref, w_ref, o_ref):
        o_ref[...] = jnp.dot(x_ref[...], w_ref[...],
                             preferred_element_type=jnp.float32)
    return pl.pallas_call(
        f, out_shape=jax.ShapeDtypeStruct((x.shape[0], W.shape[1]),
                                          jnp.float32))(x, W)


def _tc_first(p0, p1, d0, d1, b):
    def f(p0r, p1r, d0r, d1r, br, h_ref, invd_ref):
        invd = 1.0 / jnp.maximum(d0r[...] + d1r[...], 1.0)
        invd_ref[...] = invd
        h_ref[...] = jnp.maximum((p0r[...] + p1r[...]) * invd + br[...], 0.0)
    return pl.pallas_call(
        f, out_shape=[jax.ShapeDtypeStruct((N, HID), jnp.float32),
                      jax.ShapeDtypeStruct((N, 1), jnp.float32)])(
        p0, p1, d0, d1, b)


def _tc_layer(p0, p1, invd, h, W, b):
    def f(p0r, p1r, invdr, hr, wr, br, o_ref):
        agg = (p0r[...] + p1r[...]) * invdr[...]
        o_ref[...] = jnp.maximum(
            jnp.dot(agg, wr[...], preferred_element_type=jnp.float32)
            + br[...], 0.0) + hr[...]
    return pl.pallas_call(
        f, out_shape=jax.ShapeDtypeStruct((N, HID), jnp.float32))(
        p0, p1, invd, h, W, b)


def _tc_layer_last(p0, p1, invd, h, W, b, Wnext):
    def f(p0r, p1r, invdr, hr, wr, br, wnr, h_ref, y_ref):
        agg = (p0r[...] + p1r[...]) * invdr[...]
        hn = jnp.maximum(
            jnp.dot(agg, wr[...], preferred_element_type=jnp.float32)
            + br[...], 0.0) + hr[...]
        h_ref[...] = hn
        y_ref[...] = jnp.dot(hn, wnr[...], preferred_element_type=jnp.float32)
    return pl.pallas_call(
        f, out_shape=[jax.ShapeDtypeStruct((N, HID), jnp.float32),
                      jax.ShapeDtypeStruct((N, DW), jnp.float32)])(
        p0, p1, invd, h, W, b, Wnext)


def _tc_final(q0, q1, invd, b):
    def f(q0r, q1r, invdr, br, o_ref):
        o_ref[...] = (q0r[...] + q1r[...]) * invdr[...] + br[...]
    return pl.pallas_call(
        f, out_shape=jax.ShapeDtypeStruct((N, 1), jnp.float32))(
        q0, q1, invd, b)


def kernel(x, edge_index, W0, b0, W1, b1, W2, b2, W3, b3, W4, b4, W5, b5):
    row = edge_index[0].astype(jnp.int32)
    col = edge_index[1].astype(jnp.int32)
    pad = EWP - EW
    rowp = jnp.concatenate(
        [row.reshape(NW, EW),
         jnp.full((NW, pad), DUMMY, jnp.int32)], axis=1).reshape(NW, NCHUNK, CH)
    colp = jnp.concatenate(
        [col.reshape(NW, EW),
         jnp.zeros((NW, pad), jnp.int32)], axis=1).reshape(NW, NCHUNK, CH)

    z64 = jnp.zeros((RPT, HID), jnp.float32)
    z16 = jnp.zeros((RPT, DW), jnp.float32)
    ones = jnp.ones((CH, DW), jnp.float32)

    Ws = [W1, W2, W3, W4]
    bs = [b1, b2, b3, b4]
    W5p = jnp.pad(W5, ((0, 0), (0, DW - W5.shape[1])))

    y0 = _tc_matmul(x, W0)
    P0, P1, D0, D1 = _agg64_deg(y0, rowp, colp, z64, z16, ones)
    h, invd = _tc_first(P0[:N], P1[:N], D0[:N, 0:1], D1[:N, 0:1],
                        b0.reshape(1, HID))

    for i in range(4):
        Q0, Q1 = _agg64(h, rowp, colp, z64)
        if i < 3:
            h = _tc_layer(Q0[:N], Q1[:N], invd, h, Ws[i],
                          bs[i].reshape(1, HID))
        else:
            h, y5 = _tc_layer_last(Q0[:N], Q1[:N], invd, h, Ws[i],
                                   bs[i].reshape(1, HID), W5p)

    F0, F1 = _agg16(y5, rowp, colp, z16)
    out = _tc_final(F0[:N, 0:1], F1[:N, 0:1], invd,
                    b5.reshape(1, 1))
    return out[:, 0]


# speedup vs baseline: 1.0204x; 1.0204x over previous
import functools

import jax
import jax.numpy as jnp
from jax import lax
from jax.experimental import pallas as pl
from jax.experimental.pallas import tpu as pltpu
from jax.experimental.pallas import tpu_sc as plsc

N = 10000
E = 320000
HID = 64
N_LAYERS = 6

NC, NS = 2, 16
NW = NC * NS
CH = 128
EW = E // NW
NCHUNK = -(-EW // CH)
GAHEAD = 3
NBUF = 4
EWP = NCHUNK * CH
NTAB = 10240
RPT = NTAB // NS
DUMMY = N
DW = 16


def _make_agg(D, with_deg):
    out_type = [jax.ShapeDtypeStruct((NTAB, D), jnp.float32) for _ in range(NC)]
    scratch = [
        pltpu.VMEM((NCHUNK, CH), jnp.int32),
        pltpu.VMEM((NCHUNK, CH), jnp.int32),
        pltpu.VMEM((NBUF, CH, D), jnp.float32),
        pltpu.VMEM_SHARED((NTAB, D), jnp.float32),
        pltpu.SemaphoreType.DMA,
    ] + [pltpu.SemaphoreType.DMA] * (2 * NBUF)
    if with_deg:
        out_type += [jax.ShapeDtypeStruct((NTAB, DW), jnp.float32)
                     for _ in range(NC)]
        scratch += [
            pltpu.VMEM((CH, DW), jnp.float32),
            pltpu.VMEM_SHARED((NTAB, DW), jnp.float32),
            pltpu.SemaphoreType.DMA,
        ]
    mesh = plsc.VectorSubcoreMesh(core_axis_name="c", subcore_axis_name="s")

    def body(*refs):
        if with_deg:
            (tab, rowi, coli, z, zd, ones_h,
             out0, out1, dout0, dout1,
             rowi_v, coli_v, rows_v, acc, sem_in, *rest) = refs
            sg, ss = rest[:NBUF], rest[NBUF:2 * NBUF]
            ones_v, dacc, sem_d = rest[2 * NBUF:]
        else:
            (tab, rowi, coli, z,
             out0, out1,
             rowi_v, coli_v, rows_v, acc, sem_in, *rest) = refs
            sg, ss = rest[:NBUF], rest[NBUF:2 * NBUF]
        c = lax.axis_index("c")
        s = lax.axis_index("s")
        w = c * NS + s
        sl = pl.ds(s * RPT, RPT)
        cp_coli = pltpu.async_copy(coli.at[w], coli_v, sem_in)
        stage = [pltpu.async_copy(z, acc.at[sl], sem_in),
                 pltpu.async_copy(rowi.at[w], rowi_v, sem_in)]
        if with_deg:
            stage.append(pltpu.async_copy(zd, dacc.at[sl], sem_in))
            stage.append(pltpu.async_copy(ones_h, ones_v, sem_in))

        gath = [None] * NCHUNK
        scat = [None] * NCHUNK
        dscat = [None] * NCHUNK

        def fire_gather(j):
            gath[j] = pltpu.async_copy(tab.at[coli_v.at[j]],
                                       rows_v.at[j % NBUF], sg[j % NBUF])

        cp_coli.wait()
        for j in range(min(GAHEAD, NCHUNK)):
            fire_gather(j)
        for cp in stage:
            cp.wait()
        plsc.subcore_barrier()
        for j in range(NCHUNK):
            b = j % NBUF
            gath[j].wait()
            scat[j] = pltpu.async_copy(rows_v.at[b], acc.at[rowi_v.at[j]],
                                       ss[b], add=True)
            if with_deg:
                dscat[j] = pltpu.async_copy(ones_v, dacc.at[rowi_v.at[j]],
                                            sem_d, add=True)
                if j >= 4:
                    dscat[j - 4].wait()
            nxt = j + GAHEAD
            if nxt < NCHUNK:
                if nxt >= NBUF:
                    scat[nxt - NBUF].wait()
                fire_gather(nxt)
        for j in range(max(0, NCHUNK - NBUF), NCHUNK):
            scat[j].wait()
        if with_deg:
            for j in range(max(0, NCHUNK - 4), NCHUNK):
                dscat[j].wait()
        plsc.subcore_barrier()

        @pl.when(c == 0)
        def _():
            pltpu.sync_copy(acc.at[sl], out0.at[sl])
            if with_deg:
                pltpu.sync_copy(dacc.at[sl], dout0.at[sl])

        @pl.when(c == 1)
        def _():
            pltpu.sync_copy(acc.at[sl], out1.at[sl])
            if with_deg:
                pltpu.sync_copy(dacc.at[sl], dout1.at[sl])

    return pl.kernel(body, out_type=out_type, mesh=mesh,
                     scratch_types=scratch,
                     compiler_params=pltpu.CompilerParams(
                         use_tc_tiling_on_sc=False))


_agg64_deg = _make_agg(HID, True)
_agg64 = _make_agg(HID, False)
_agg16 = _make_agg(DW, False)


def _tc_matmul(x, W):
    def f(x_ref, w_ref, o_ref):
        o_ref[...] = jnp.dot(x_ref[...], w_ref[...],
                             preferred_element_type=jnp.float32)
    return pl.pallas_call(
        f, out_shape=jax.ShapeDtypeStruct((x.shape[0], W.shape[1]),
                                          jnp.float32))(x, W)


def _tc_first(p0, p1, d0, d1, b):
    def f(p0r, p1r, d0r, d1r, br, h_ref, invd_ref):
        invd = 1.0 / jnp.maximum(d0r[...] + d1r[...], 1.0)
        invd_ref[...] = invd
        h_ref[...] = jnp.maximum((p0r[...] + p1r[...]) * invd + br[...], 0.0)
    return pl.pallas_call(
        f, out_shape=[jax.ShapeDtypeStruct((N, HID), jnp.float32),
                      jax.ShapeDtypeStruct((N, 1), jnp.float32)])(
        p0, p1, d0, d1, b)


def _tc_layer(p0, p1, invd, h, W, b):
    def f(p0r, p1r, invdr, hr, wr, br, o_ref):
        agg = (p0r[...] + p1r[...]) * invdr[...]
        o_ref[...] = jnp.maximum(
            jnp.dot(agg, wr[...], preferred_element_type=jnp.float32)
            + br[...], 0.0) + hr[...]
    return pl.pallas_call(
        f, out_shape=jax.ShapeDtypeStruct((N, HID), jnp.float32))(
        p0, p1, invd, h, W, b)


def _tc_layer_last(p0, p1, invd, h, W, b, Wnext):
    def f(p0r, p1r, invdr, hr, wr, br, wnr, h_ref, y_ref):
        agg = (p0r[...] + p1r[...]) * invdr[...]
        hn = jnp.maximum(
            jnp.dot(agg, wr[...], preferred_element_type=jnp.float32)
            + br[...], 0.0) + hr[...]
        h_ref[...] = hn
        y_ref[...] = jnp.dot(hn, wnr[...], preferred_element_type=jnp.float32)
    return pl.pallas_call(
        f, out_shape=[jax.ShapeDtypeStruct((N, HID), jnp.float32),
                      jax.ShapeDtypeStruct((N, DW), jnp.float32)])(
        p0, p1, invd, h, W, b, Wnext)


def _tc_final(q0, q1, invd, b):
    def f(q0r, q1r, invdr, br, o_ref):
        o_ref[...] = (q0r[...] + q1r[...]) * invdr[...] + br[...]
    return pl.pallas_call(
        f, out_shape=jax.ShapeDtypeStruct((N, 1), jnp.float32))(
        q0, q1, invd, b)


def kernel(x, edge_index, W0, b0, W1, b1, W2, b2, W3, b3, W4, b4, W5, b5):
    row = edge_index[0].astype(jnp.int32)
    col = edge_index[1].astype(jnp.int32)
    pad = EWP - EW
    rowp = jnp.concatenate(
        [row.reshape(NW, EW),
         jnp.full((NW, pad), DUMMY, jnp.int32)], axis=1).reshape(NW, NCHUNK, CH)
    colp = jnp.concatenate(
        [col.reshape(NW, EW),
         jnp.zeros((NW, pad), jnp.int32)], axis=1).reshape(NW, NCHUNK, CH)

    z64 = jnp.zeros((RPT, HID), jnp.float32)
    z16 = jnp.zeros((RPT, DW), jnp.float32)
    ones = jnp.ones((CH, DW), jnp.float32)

    Ws = [W1, W2, W3, W4]
    bs = [b1, b2, b3, b4]
    W5p = jnp.pad(W5, ((0, 0), (0, DW - W5.shape[1])))

    y0 = _tc_matmul(x, W0)
    P0, P1, D0, D1 = _agg64_deg(y0, rowp, colp, z64, z16, ones)
    h, invd = _tc_first(P0[:N], P1[:N], D0[:N, 0:1], D1[:N, 0:1],
                        b0.reshape(1, HID))

    for i in range(4):
        Q0, Q1 = _agg64(h, rowp, colp, z64)
        if i < 3:
            h = _tc_layer(Q0[:N], Q1[:N], invd, h, Ws[i],
                          bs[i].reshape(1, HID))
        else:
            h, y5 = _tc_layer_last(Q0[:N], Q1[:N], invd, h, Ws[i],
                                   bs[i].reshape(1, HID), W5p)

    F0, F1 = _agg16(y5, rowp, colp, z16)
    out = _tc_final(F0[:N, 0:1], F1[:N, 0:1], invd,
                    b5.reshape(1, 1))
    return out[:, 0]
